# baseline (speedup 1.0000x reference)
import functools

import jax
import jax.numpy as jnp
from jax import lax
from jax.experimental import pallas as pl
from jax.experimental.pallas import tpu as pltpu
from jax.experimental.pallas import tpu_sc as plsc

N = 32768
H = 4096
E = 64
K = 8

BM = 512

CHUNKS = (8192, 8192, 8192, 8192)


def _scores_tc_kernel(acc_ref, x_ref, wt_ref, b_ref, out2d_ref, outc_ref):
    del acc_ref
    logits = jnp.dot(x_ref[...], wt_ref[...], preferred_element_type=jnp.float32)
    scores = jax.nn.sigmoid(logits + b_ref[...])
    out2d_ref[...] = scores
    outc_ref[...] = scores


def _scores_tc(scores_acc, x, wt, b2d, chunk_rows, row0):
    blk0 = row0 // BM
    return pl.pallas_call(
        _scores_tc_kernel,
        grid=(chunk_rows // BM,),
        in_specs=[
            pl.BlockSpec(memory_space=pltpu.MemorySpace.HBM),
            pl.BlockSpec((BM, H), lambda i: (blk0 + i, 0)),
            pl.BlockSpec((H, E), lambda i: (0, 0)),
            pl.BlockSpec((1, E), lambda i: (0, 0)),
        ],
        out_specs=[
            pl.BlockSpec((BM, E), lambda i: (blk0 + i, 0)),
            pl.BlockSpec((BM, E), lambda i: (i, 0)),
        ],
        out_shape=[
            jax.ShapeDtypeStruct((N, E), jnp.float32),
            jax.ShapeDtypeStruct((chunk_rows, E), jnp.float32),
        ],
        input_output_aliases={0: 0},
    )(scores_acc, x, wt, b2d)


def _topk_sc_body(nt, row0, scores_hbm, w_hbm, i_hbm, sbuf, wbuf, ibuf):
    info = plsc.get_sparse_core_info()
    nc, ns = info.num_cores, info.num_subcores
    nw = nc * ns
    tpw = nt // nw

    wid = lax.axis_index("s") * nc + lax.axis_index("c")
    sbase = wid * (tpw * E)
    obase = (row0 + wid * tpw) * K

    pltpu.sync_copy(scores_hbm.at[pl.ds(sbase, tpw * E)], sbuf)

    lane = lax.iota(jnp.int32, 16)
    neg1 = jnp.full((16,), -1.0, jnp.float32)
    zero_i = jnp.zeros((16,), jnp.int32)
    e_vecs = [jnp.full((16,), e, jnp.int32) for e in range(E)]

    def group_body(g, carry):
        gather_base = g * (16 * E) + lane * E
        tv = [neg1] * K
        ti = [zero_i] * K
        for e in range(E):
            v = plsc.load_gather(sbuf, [gather_base + e])
            vi = e_vecs[e]
            for j in range(K):
                gt = v > tv[j]
                nv = jnp.where(gt, v, tv[j])
                ni = jnp.where(gt, vi, ti[j])
                cv = jnp.where(gt, tv[j], v)
                ci = jnp.where(gt, ti[j], vi)
                tv[j], ti[j] = nv, ni
                v, vi = cv, ci

        denom = tv[0]
        for j in range(1, K):
            denom = denom + tv[j]
        recip = 1.0 / jnp.maximum(denom, 1e-12)

        out_base = g * (16 * K) + lane * K
        for j in range(K):
            plsc.store_scatter(wbuf, [out_base + j], tv[j] * recip)
            plsc.store_scatter(ibuf, [out_base + j], ti[j])
        return carry

    lax.fori_loop(0, tpw // 16, group_body, 0)

    pltpu.sync_copy(wbuf, w_hbm.at[pl.ds(obase, tpw * K)])
    pltpu.sync_copy(ibuf, i_hbm.at[pl.ds(obase, tpw * K)])


def _topk_sc(scores_flat, w_ref, i_ref, nt, row0):
    info = plsc.get_sparse_core_info()
    nw = info.num_cores * info.num_subcores
    tpw = nt // nw
    pl.kernel(
        functools.partial(_topk_sc_body, nt, row0),
        out_type=[],
        mesh=plsc.VectorSubcoreMesh(core_axis_name="c", subcore_axis_name="s"),
        compiler_params=pltpu.CompilerParams(needs_layout_passes=False),
        scratch_types=[
            pltpu.VMEM((tpw * E,), jnp.float32),
            pltpu.VMEM((tpw * K,), jnp.float32),
            pltpu.VMEM((tpw * K,), jnp.int32),
        ],
    )(scores_flat, w_ref, i_ref)


def kernel(x, W, expert_bias):
    wt = W.T
    b2d = expert_bias.reshape(1, E)
    scores_acc = jnp.zeros((N, E), jnp.float32)
    w_ref = jax.new_ref(jnp.zeros((N * K,), jnp.float32))
    i_ref = jax.new_ref(jnp.zeros((N * K,), jnp.int32))
    row0 = 0
    for nt in CHUNKS:
        scores_acc, chunk2d = _scores_tc(scores_acc, x, wt, b2d, nt, row0)
        _topk_sc(chunk2d.reshape(nt * E), w_ref, i_ref, nt, row0)
        row0 += nt
    weights = w_ref[...].reshape(N, K)
    top_idx = i_ref[...].reshape(N, K).astype(jnp.int64)
    return (weights, top_idx, scores_acc)


# speedup vs baseline: 1.0870x; 1.0870x over previous
import functools

import jax
import jax.numpy as jnp
from jax import lax
from jax.experimental import pallas as pl
from jax.experimental.pallas import tpu as pltpu
from jax.experimental.pallas import tpu_sc as plsc

N = 32768
H = 4096
E = 64
K = 8

BM = 512


def _scores_tc_kernel(x_ref, wt_ref, b_ref, out_ref):
    logits = jnp.dot(x_ref[...], wt_ref[...], preferred_element_type=jnp.float32)
    out_ref[...] = jax.nn.sigmoid(logits + b_ref[...])


def _scores_tc(x, wt, b2d, chunk_rows, row0):
    blk0 = row0 // BM
    return pl.pallas_call(
        _scores_tc_kernel,
        grid=(chunk_rows // BM,),
        in_specs=[
            pl.BlockSpec((BM, H), lambda i: (blk0 + i, 0)),
            pl.BlockSpec((H, E), lambda i: (0, 0)),
            pl.BlockSpec((1, E), lambda i: (0, 0)),
        ],
        out_specs=pl.BlockSpec((BM, E), lambda i: (i, 0)),
        out_shape=jax.ShapeDtypeStruct((chunk_rows, E), jnp.float32),
    )(x, wt, b2d)


def _topk_sc_body(nt, scores_hbm, w_hbm, i_hbm, sbuf, wbuf, ibuf):
    info = plsc.get_sparse_core_info()
    nc, ns = info.num_cores, info.num_subcores
    nw = nc * ns
    tpw = nt // nw

    wid = lax.axis_index("s") * nc + lax.axis_index("c")
    obase = wid * (tpw * K)

    pltpu.sync_copy(scores_hbm.at[pl.ds(wid * tpw, tpw)], sbuf)

    lane = lax.iota(jnp.int32, 16)
    neg1 = jnp.full((16,), -1.0, jnp.float32)
    zero_i = jnp.zeros((16,), jnp.int32)
    e_vecs = [jnp.full((16,), e, jnp.int32) for e in range(E)]

    def group_body(g, carry):
        rows = g * 16 + lane
        tv = [neg1] * K
        ti = [zero_i] * K
        for e in range(E):
            v = plsc.load_gather(sbuf, [rows, e_vecs[e]])
            vi = e_vecs[e]
            for j in range(K):
                gt = v > tv[j]
                nv = jnp.where(gt, v, tv[j])
                ni = jnp.where(gt, vi, ti[j])
                cv = jnp.where(gt, tv[j], v)
                ci = jnp.where(gt, ti[j], vi)
                tv[j], ti[j] = nv, ni
                v, vi = cv, ci

        denom = tv[0]
        for j in range(1, K):
            denom = denom + tv[j]
        recip = 1.0 / jnp.maximum(denom, 1e-12)

        out_base = g * (16 * K) + lane * K
        for j in range(K):
            plsc.store_scatter(wbuf, [out_base + j], tv[j] * recip)
            plsc.store_scatter(ibuf, [out_base + j], ti[j])
        return carry

    lax.fori_loop(0, tpw // 16, group_body, 0)

    pltpu.sync_copy(wbuf, w_hbm.at[pl.ds(obase, tpw * K)])
    pltpu.sync_copy(ibuf, i_hbm.at[pl.ds(obase, tpw * K)])


def _topk_sc(scores_flat, nt):
    info = plsc.get_sparse_core_info()
    nw = info.num_cores * info.num_subcores
    tpw = nt // nw
    return pl.kernel(
        functools.partial(_topk_sc_body, nt),
        out_type=[
            jax.ShapeDtypeStruct((nt * K,), jnp.float32),
            jax.ShapeDtypeStruct((nt * K,), jnp.int32),
        ],
        mesh=plsc.VectorSubcoreMesh(core_axis_name="c", subcore_axis_name="s"),
        compiler_params=pltpu.CompilerParams(needs_layout_passes=False),
        scratch_types=[
            pltpu.VMEM((tpw, E), jnp.float32),
            pltpu.VMEM((tpw * K,), jnp.float32),
            pltpu.VMEM((tpw * K,), jnp.int32),
        ],
    )(scores_flat)


CHUNKS = (8192, 8192, 8192, 8192)


def kernel(x, W, expert_bias):
    wt = W.T
    b2d = expert_bias.reshape(1, E)
    row0s = [sum(CHUNKS[:c]) for c in range(len(CHUNKS))]
    scores_c = [
        _scores_tc(x, wt, b2d, nt, r0) for nt, r0 in zip(CHUNKS, row0s)
    ]
    topk_c = [_topk_sc(s, nt) for s, nt in zip(scores_c, CHUNKS)]
    scores = jnp.concatenate(scores_c, axis=0)
    weights = jnp.concatenate([w for w, _ in topk_c], axis=0).reshape(N, K)
    top_idx = (
        jnp.concatenate([i for _, i in topk_c], axis=0)
        .reshape(N, K)
        .astype(jnp.int64)
    )
    return (weights, top_idx, scores)
